# baseline (speedup 1.0000x reference)
import jax
import jax.numpy as jnp
from jax.experimental import pallas as pl
from jax.experimental.pallas import tpu as pltpu


def _add_body(s_ref, x_ref, o_ref):
    o_ref[...] = x_ref[...] + s_ref[0]


def kernel(input, pad, channels_last):
    x = input
    s = (jnp.asarray(pad, x.dtype) + jnp.asarray(channels_last, x.dtype)).reshape(1)
    lanes = x.shape[-1] * x.shape[-2]
    rows = x.size // lanes
    bm = 128
    while rows % bm:
        bm //= 2
    x2 = x.reshape(rows, lanes)
    out = pl.pallas_call(
        _add_body,
        grid=(rows // bm,),
        in_specs=[
            pl.BlockSpec(memory_space=pltpu.SMEM),
            pl.BlockSpec((bm, lanes), lambda i: (i, 0)),
        ],
        out_specs=pl.BlockSpec((bm, lanes), lambda i: (i, 0)),
        out_shape=jax.ShapeDtypeStruct((rows, lanes), x.dtype),
        compiler_params=pltpu.CompilerParams(
            dimension_semantics=("arbitrary",),
        ),
    )(s, x2)
    return out.reshape(x.shape)
